# baseline (speedup 1.0000x reference)
import jax
import jax.numpy as jnp
from jax.experimental import pallas as pl
from jax.experimental.pallas import tpu as pltpu

B = 128
CLS = 8
POOL = 1024
HALF = 6
D = 512
SEQ = 77
SUF = SEQ - 1 - 2 * HALF


def _body(idx_ref, g0, g1, a0, a1, pref, suf, ncpref, ncsuf, gid, aid,
          tokr, nctokr, out_p, out_tok, out_ncp, out_nctok):
    b = pl.program_id(1)
    is_g = b < (B // 2)
    r0 = jnp.where(is_g, g0[0], a0[0])
    r1 = jnp.where(is_g, g1[0], a1[0])
    out_p[0, 0:1, :] = pref[0]
    out_p[0, 1:1 + HALF, :] = r0
    out_p[0, 1 + HALF:1 + 2 * HALF, :] = r1
    out_p[0, 1 + 2 * HALF:SEQ, :] = suf[0]
    out_tok[0] = tokr[0]
    out_ncp[0, 0:1, :] = ncpref[0]
    out_ncp[0, 1:1 + HALF, :] = gid[0]
    out_ncp[0, 1 + HALF:1 + 2 * HALF, :] = aid[0]
    out_ncp[0, 1 + 2 * HALF:SEQ, :] = ncsuf[0]
    out_nctok[0] = nctokr[0]


def kernel(indices_g, global_prompt, attribute_prompt, token_prefix,
           token_suffix, nc_token_prefix, nc_token_suffix,
           tokenized_prompts, nc_tokenized_prompts):
    tok3 = tokenized_prompts.reshape(CLS, 1, SEQ)
    nctok3 = nc_tokenized_prompts.reshape(1, 1, SEQ)

    grid = (CLS, B)

    in_specs = [
        pl.BlockSpec((1, HALF, D), lambda c, b, idx: (idx[(2 * b) % B], 0, 0)),
        pl.BlockSpec((1, HALF, D), lambda c, b, idx: (idx[(2 * b + 1) % B], 0, 0)),
        pl.BlockSpec((1, HALF, D), lambda c, b, idx: (idx[(2 * b) % B], 0, 0)),
        pl.BlockSpec((1, HALF, D), lambda c, b, idx: (idx[(2 * b + 1) % B], 0, 0)),
        pl.BlockSpec((1, 1, D), lambda c, b, idx: (c, 0, 0)),
        pl.BlockSpec((1, SUF, D), lambda c, b, idx: (c, 0, 0)),
        pl.BlockSpec((1, 1, D), lambda c, b, idx: (0, 0, 0)),
        pl.BlockSpec((1, SUF, D), lambda c, b, idx: (0, 0, 0)),
        pl.BlockSpec((1, HALF, D), lambda c, b, idx: (c * B + b, 0, 0)),
        pl.BlockSpec((1, HALF, D), lambda c, b, idx: (c * B + b, 0, 0)),
        pl.BlockSpec((1, 1, SEQ), lambda c, b, idx: (c, 0, 0)),
        pl.BlockSpec((1, 1, SEQ), lambda c, b, idx: (0, 0, 0)),
    ]
    out_specs = [
        pl.BlockSpec((1, SEQ, D), lambda c, b, idx: (b * CLS + c, 0, 0)),
        pl.BlockSpec((1, 1, SEQ), lambda c, b, idx: (b * CLS + c, 0, 0)),
        pl.BlockSpec((1, SEQ, D), lambda c, b, idx: (c * B + b, 0, 0)),
        pl.BlockSpec((1, 1, SEQ), lambda c, b, idx: (c * B + b, 0, 0)),
    ]
    out_shape = [
        jax.ShapeDtypeStruct((B * CLS, SEQ, D), jnp.float32),
        jax.ShapeDtypeStruct((B * CLS, 1, SEQ), jnp.int32),
        jax.ShapeDtypeStruct((POOL, SEQ, D), jnp.float32),
        jax.ShapeDtypeStruct((POOL, 1, SEQ), jnp.int32),
    ]

    grid_spec = pltpu.PrefetchScalarGridSpec(
        num_scalar_prefetch=1,
        grid=grid,
        in_specs=in_specs,
        out_specs=out_specs,
    )
    prompts, tok3o, nc_prompts, nctok3o = pl.pallas_call(
        _body,
        grid_spec=grid_spec,
        out_shape=out_shape,
    )(indices_g, global_prompt, global_prompt, attribute_prompt,
      attribute_prompt, token_prefix, token_suffix, nc_token_prefix,
      nc_token_suffix, global_prompt, attribute_prompt, tok3, nctok3)

    return (prompts, tok3o.reshape(B * CLS, SEQ), nc_prompts,
            nctok3o.reshape(POOL, SEQ))


# speedup vs baseline: 2.1267x; 2.1267x over previous
import jax
import jax.numpy as jnp
from jax.experimental import pallas as pl
from jax.experimental.pallas import tpu as pltpu

B = 128
CLS = 8
POOL = 1024
HALF = 6
D = 512
SEQ = 77
SUF = SEQ - 1 - 2 * HALF


def _body(idx_ref, g0, g1, a0, a1, pref, suf, ncpref, ncsuf, gid, aid,
          tokr, nctokr, out_p, out_tok, out_ncp, out_nctok):
    b = pl.program_id(0)
    is_g = b < (B // 2)
    r0 = jnp.where(is_g, g0[0], a0[0])
    r1 = jnp.where(is_g, g1[0], a1[0])
    ctx = jnp.concatenate([r0, r1], axis=0)
    out_p[:, 0:1, :] = pref[:]
    out_p[:, 1:1 + 2 * HALF, :] = jnp.broadcast_to(ctx[None], (CLS, 2 * HALF, D))
    out_p[:, 1 + 2 * HALF:SEQ, :] = suf[:]
    out_tok[:] = tokr[:]
    out_ncp[:, 0:1, :] = jnp.broadcast_to(ncpref[:], (CLS, 1, D))
    out_ncp[:, 1:1 + HALF, :] = gid[:]
    out_ncp[:, 1 + HALF:1 + 2 * HALF, :] = aid[:]
    out_ncp[:, 1 + 2 * HALF:SEQ, :] = jnp.broadcast_to(ncsuf[:], (CLS, SUF, D))
    out_nctok[:] = jnp.broadcast_to(nctokr[:], (CLS, SEQ))


def kernel(indices_g, global_prompt, attribute_prompt, token_prefix,
           token_suffix, nc_token_prefix, nc_token_suffix,
           tokenized_prompts, nc_tokenized_prompts):
    grid = (B,)

    in_specs = [
        pl.BlockSpec((1, HALF, D), lambda b, idx: (idx[(2 * b) % B], 0, 0)),
        pl.BlockSpec((1, HALF, D), lambda b, idx: (idx[(2 * b + 1) % B], 0, 0)),
        pl.BlockSpec((1, HALF, D), lambda b, idx: (idx[(2 * b) % B], 0, 0)),
        pl.BlockSpec((1, HALF, D), lambda b, idx: (idx[(2 * b + 1) % B], 0, 0)),
        pl.BlockSpec((CLS, 1, D), lambda b, idx: (0, 0, 0)),
        pl.BlockSpec((CLS, SUF, D), lambda b, idx: (0, 0, 0)),
        pl.BlockSpec((1, 1, D), lambda b, idx: (0, 0, 0)),
        pl.BlockSpec((1, SUF, D), lambda b, idx: (0, 0, 0)),
        pl.BlockSpec((CLS, HALF, D), lambda b, idx: (b, 0, 0)),
        pl.BlockSpec((CLS, HALF, D), lambda b, idx: (b, 0, 0)),
        pl.BlockSpec((CLS, SEQ), lambda b, idx: (0, 0)),
        pl.BlockSpec((1, SEQ), lambda b, idx: (0, 0)),
    ]
    out_specs = [
        pl.BlockSpec((CLS, SEQ, D), lambda b, idx: (b, 0, 0)),
        pl.BlockSpec((CLS, SEQ), lambda b, idx: (b, 0)),
        pl.BlockSpec((CLS, SEQ, D), lambda b, idx: (b, 0, 0)),
        pl.BlockSpec((CLS, SEQ), lambda b, idx: (b, 0)),
    ]
    out_shape = [
        jax.ShapeDtypeStruct((B * CLS, SEQ, D), jnp.float32),
        jax.ShapeDtypeStruct((B * CLS, SEQ), jnp.int32),
        jax.ShapeDtypeStruct((POOL, SEQ, D), jnp.float32),
        jax.ShapeDtypeStruct((POOL, SEQ), jnp.int32),
    ]

    grid_spec = pltpu.PrefetchScalarGridSpec(
        num_scalar_prefetch=1,
        grid=grid,
        in_specs=in_specs,
        out_specs=out_specs,
    )
    prompts, tok, nc_prompts, nc_tok = pl.pallas_call(
        _body,
        grid_spec=grid_spec,
        out_shape=out_shape,
    )(indices_g, global_prompt, global_prompt, attribute_prompt,
      attribute_prompt, token_prefix, token_suffix, nc_token_prefix,
      nc_token_suffix, global_prompt, attribute_prompt, tokenized_prompts,
      nc_tokenized_prompts)

    return (prompts, tok, nc_prompts, nc_tok)


# speedup vs baseline: 2.1497x; 1.0108x over previous
import functools
import jax
import jax.numpy as jnp
from jax.experimental import pallas as pl
from jax.experimental.pallas import tpu as pltpu

B = 128
CLS = 8
POOL = 1024
HALF = 6
D = 512
SEQ = 77
HEAD = 1 + 2 * HALF
SUF = SEQ - HEAD
CHUNK = 2


def _streams(b, slot, sp, snc, tokr, nctok_scr, out_p, out_tok, out_ncp,
             out_nctok):
    r0 = CLS * b
    st = []
    for j in range(0, CLS, CHUNK):
        st.append((sp.at[slot, pl.ds(j, CHUNK)],
                   out_p.at[pl.ds(r0 + j, CHUNK), :, :]))
        st.append((snc.at[slot, pl.ds(j, CHUNK)],
                   out_ncp.at[pl.ds(r0 + j, CHUNK), :, :]))
    st.append((tokr, out_tok.at[pl.ds(r0, CLS), :]))
    st.append((nctok_scr, out_nctok.at[pl.ds(r0, CLS), :]))
    return st


N_STREAMS = 2 * (CLS // CHUNK) + 2


def _body(idx_ref, g0, g1, a0, a1, pref, suf, ncpref, ncsuf, gid, aid,
          tokr, nctokr, out_p, out_tok, out_ncp, out_nctok,
          sp, snc, nctok_scr, sems):
    b = pl.program_id(0)
    slot = jax.lax.rem(b, 2)

    @pl.when(b == 0)
    def _init():
        for s in (0, 1):
            sp[s, :, 0:1, :] = pref[:]
            sp[s, :, HEAD:SEQ, :] = suf[:]
            snc[s, :, 0:1, :] = jnp.broadcast_to(ncpref[:], (CLS, 1, D))
            snc[s, :, HEAD:SEQ, :] = jnp.broadcast_to(ncsuf[:], (CLS, SUF, D))
        nctok_scr[:] = jnp.broadcast_to(nctokr[:], (CLS, SEQ))

    is_g = b < (B // 2)
    r0v = jnp.where(is_g, g0[0], a0[0])
    r1v = jnp.where(is_g, g1[0], a1[0])
    ctx = jnp.concatenate([r0v, r1v], axis=0)
    sp[slot, :, 1:HEAD, :] = jnp.broadcast_to(ctx[None], (CLS, 2 * HALF, D))
    snc[slot, :, 1:1 + HALF, :] = gid[:]
    snc[slot, :, 1 + HALF:HEAD, :] = aid[:]

    mk = functools.partial(_streams, sp=sp, snc=snc, tokr=tokr,
                           nctok_scr=nctok_scr, out_p=out_p, out_tok=out_tok,
                           out_ncp=out_ncp, out_nctok=out_nctok)

    for k, (src, dst) in enumerate(mk(b, slot)):
        pltpu.make_async_copy(src, dst, sems.at[slot, k]).start()

    @pl.when(b > 0)
    def _wait_prev():
        for k, (src, dst) in enumerate(mk(b - 1, 1 - slot)):
            pltpu.make_async_copy(src, dst, sems.at[1 - slot, k]).wait()

    @pl.when(b == B - 1)
    def _wait_own():
        for k, (src, dst) in enumerate(mk(b, slot)):
            pltpu.make_async_copy(src, dst, sems.at[slot, k]).wait()


def kernel(indices_g, global_prompt, attribute_prompt, token_prefix,
           token_suffix, nc_token_prefix, nc_token_suffix,
           tokenized_prompts, nc_tokenized_prompts):
    grid = (B,)

    in_specs = [
        pl.BlockSpec((1, HALF, D), lambda b, idx: (idx[(2 * b) % B], 0, 0)),
        pl.BlockSpec((1, HALF, D), lambda b, idx: (idx[(2 * b + 1) % B], 0, 0)),
        pl.BlockSpec((1, HALF, D), lambda b, idx: (idx[(2 * b) % B], 0, 0)),
        pl.BlockSpec((1, HALF, D), lambda b, idx: (idx[(2 * b + 1) % B], 0, 0)),
        pl.BlockSpec((CLS, 1, D), lambda b, idx: (0, 0, 0)),
        pl.BlockSpec((CLS, SUF, D), lambda b, idx: (0, 0, 0)),
        pl.BlockSpec((1, 1, D), lambda b, idx: (0, 0, 0)),
        pl.BlockSpec((1, SUF, D), lambda b, idx: (0, 0, 0)),
        pl.BlockSpec((CLS, HALF, D), lambda b, idx: (b, 0, 0)),
        pl.BlockSpec((CLS, HALF, D), lambda b, idx: (b, 0, 0)),
        pl.BlockSpec((CLS, SEQ), lambda b, idx: (0, 0)),
        pl.BlockSpec((1, SEQ), lambda b, idx: (0, 0)),
    ]
    out_specs = [
        pl.BlockSpec(memory_space=pl.ANY),
        pl.BlockSpec(memory_space=pl.ANY),
        pl.BlockSpec(memory_space=pl.ANY),
        pl.BlockSpec(memory_space=pl.ANY),
    ]
    out_shape = [
        jax.ShapeDtypeStruct((B * CLS, SEQ, D), jnp.float32),
        jax.ShapeDtypeStruct((B * CLS, SEQ), jnp.int32),
        jax.ShapeDtypeStruct((POOL, SEQ, D), jnp.float32),
        jax.ShapeDtypeStruct((POOL, SEQ), jnp.int32),
    ]
    scratch_shapes = [
        pltpu.VMEM((2, CLS, SEQ, D), jnp.float32),
        pltpu.VMEM((2, CLS, SEQ, D), jnp.float32),
        pltpu.VMEM((CLS, SEQ), jnp.int32),
        pltpu.SemaphoreType.DMA((2, N_STREAMS)),
    ]

    grid_spec = pltpu.PrefetchScalarGridSpec(
        num_scalar_prefetch=1,
        grid=grid,
        in_specs=in_specs,
        out_specs=out_specs,
        scratch_shapes=scratch_shapes,
    )
    prompts, tok, nc_prompts, nc_tok = pl.pallas_call(
        _body,
        grid_spec=grid_spec,
        out_shape=out_shape,
    )(indices_g, global_prompt, global_prompt, attribute_prompt,
      attribute_prompt, token_prefix, token_suffix, nc_token_prefix,
      nc_token_suffix, global_prompt, attribute_prompt, tokenized_prompts,
      nc_tokenized_prompts)

    return (prompts, tok, nc_prompts, nc_tok)


# speedup vs baseline: 2.3126x; 1.0758x over previous
import jax
import jax.numpy as jnp
from jax.experimental import pallas as pl
from jax.experimental.pallas import tpu as pltpu

B = 128
CLS = 8
POOL = 1024
HALF = 6
D = 512
SEQ = 77
HEAD = 1 + 2 * HALF
SUF = SEQ - HEAD
BPS = 2
ROWS = BPS * CLS


def _body(idx_ref, *refs):
    gathers = refs[:4 * BPS]
    (pref, suf, ncpref, ncsuf, gid, aid, tokr, nctokr,
     out_p, out_tok, out_ncp, out_nctok) = refs[4 * BPS:]

    s = pl.program_id(0)
    for m in range(BPS):
        b = s * BPS + m
        g0, g1 = gathers[2 * m], gathers[2 * m + 1]
        a0, a1 = gathers[2 * BPS + 2 * m], gathers[2 * BPS + 2 * m + 1]
        is_g = b < (B // 2)
        r0v = jnp.where(is_g, g0[0], a0[0])
        r1v = jnp.where(is_g, g1[0], a1[0])
        ctx = jnp.concatenate([r0v, r1v], axis=0)
        lo = m * CLS
        out_p[lo:lo + CLS, 0:1, :] = pref[:]
        out_p[lo:lo + CLS, 1:HEAD, :] = jnp.broadcast_to(ctx[None],
                                                         (CLS, 2 * HALF, D))
        out_p[lo:lo + CLS, HEAD:SEQ, :] = suf[:]
        out_tok[lo:lo + CLS, :] = tokr[:]
        out_nctok[lo:lo + CLS, :] = jnp.broadcast_to(nctokr[:], (CLS, SEQ))
    out_ncp[:, 0:1, :] = jnp.broadcast_to(ncpref[:], (ROWS, 1, D))
    out_ncp[:, 1:1 + HALF, :] = gid[:]
    out_ncp[:, 1 + HALF:HEAD, :] = aid[:]
    out_ncp[:, HEAD:SEQ, :] = jnp.broadcast_to(ncsuf[:], (ROWS, SUF, D))


def kernel(indices_g, global_prompt, attribute_prompt, token_prefix,
           token_suffix, nc_token_prefix, nc_token_suffix,
           tokenized_prompts, nc_tokenized_prompts):
    grid = (B // BPS,)

    def gspec(m):
        return pl.BlockSpec(
            (1, HALF, D),
            lambda s, idx, m=m: (idx[(2 * BPS * s + m) % B], 0, 0))

    in_specs = (
        [gspec(m) for m in range(2 * BPS)] +
        [gspec(m) for m in range(2 * BPS)] +
        [
            pl.BlockSpec((CLS, 1, D), lambda s, idx: (0, 0, 0)),
            pl.BlockSpec((CLS, SUF, D), lambda s, idx: (0, 0, 0)),
            pl.BlockSpec((1, 1, D), lambda s, idx: (0, 0, 0)),
            pl.BlockSpec((1, SUF, D), lambda s, idx: (0, 0, 0)),
            pl.BlockSpec((ROWS, HALF, D), lambda s, idx: (s, 0, 0)),
            pl.BlockSpec((ROWS, HALF, D), lambda s, idx: (s, 0, 0)),
            pl.BlockSpec((CLS, SEQ), lambda s, idx: (0, 0)),
            pl.BlockSpec((1, SEQ), lambda s, idx: (0, 0)),
        ])
    out_specs = [
        pl.BlockSpec((ROWS, SEQ, D), lambda s, idx: (s, 0, 0)),
        pl.BlockSpec((ROWS, SEQ), lambda s, idx: (s, 0)),
        pl.BlockSpec((ROWS, SEQ, D), lambda s, idx: (s, 0, 0)),
        pl.BlockSpec((ROWS, SEQ), lambda s, idx: (s, 0)),
    ]
    out_shape = [
        jax.ShapeDtypeStruct((B * CLS, SEQ, D), jnp.float32),
        jax.ShapeDtypeStruct((B * CLS, SEQ), jnp.int32),
        jax.ShapeDtypeStruct((POOL, SEQ, D), jnp.float32),
        jax.ShapeDtypeStruct((POOL, SEQ), jnp.int32),
    ]

    grid_spec = pltpu.PrefetchScalarGridSpec(
        num_scalar_prefetch=1,
        grid=grid,
        in_specs=in_specs,
        out_specs=out_specs,
    )
    prompts, tok, nc_prompts, nc_tok = pl.pallas_call(
        _body,
        grid_spec=grid_spec,
        out_shape=out_shape,
    )(indices_g,
      *([global_prompt] * (2 * BPS)), *([attribute_prompt] * (2 * BPS)),
      token_prefix, token_suffix, nc_token_prefix, nc_token_suffix,
      global_prompt, attribute_prompt, tokenized_prompts,
      nc_tokenized_prompts)

    return (prompts, tok, nc_prompts, nc_tok)


# speedup vs baseline: 2.3154x; 1.0012x over previous
import jax
import jax.numpy as jnp
from jax.experimental import pallas as pl
from jax.experimental.pallas import tpu as pltpu

B = 128
CLS = 8
POOL = 1024
HALF = 6
D = 512
SEQ = 77
HEAD = 1 + 2 * HALF
SUF = SEQ - HEAD
BPS = 4
ROWS = BPS * CLS


def _body(idx_ref, *refs):
    gathers = refs[:4 * BPS]
    (pref, suf, ncpref, ncsuf, gid, aid, tokr, nctokr,
     out_p, out_tok, out_ncp, out_nctok) = refs[4 * BPS:]

    s = pl.program_id(0)
    for m in range(BPS):
        b = s * BPS + m
        g0, g1 = gathers[2 * m], gathers[2 * m + 1]
        a0, a1 = gathers[2 * BPS + 2 * m], gathers[2 * BPS + 2 * m + 1]
        is_g = b < (B // 2)
        r0v = jnp.where(is_g, g0[0], a0[0])
        r1v = jnp.where(is_g, g1[0], a1[0])
        ctx = jnp.concatenate([r0v, r1v], axis=0)
        lo = m * CLS
        out_p[lo:lo + CLS, 0:1, :] = pref[:]
        out_p[lo:lo + CLS, 1:HEAD, :] = jnp.broadcast_to(ctx[None],
                                                         (CLS, 2 * HALF, D))
        out_p[lo:lo + CLS, HEAD:SEQ, :] = suf[:]
        out_tok[lo:lo + CLS, :] = tokr[:]
        out_nctok[lo:lo + CLS, :] = jnp.broadcast_to(nctokr[:], (CLS, SEQ))
    out_ncp[:, 0:1, :] = jnp.broadcast_to(ncpref[:], (ROWS, 1, D))
    out_ncp[:, 1:1 + HALF, :] = gid[:]
    out_ncp[:, 1 + HALF:HEAD, :] = aid[:]
    out_ncp[:, HEAD:SEQ, :] = jnp.broadcast_to(ncsuf[:], (ROWS, SUF, D))


def kernel(indices_g, global_prompt, attribute_prompt, token_prefix,
           token_suffix, nc_token_prefix, nc_token_suffix,
           tokenized_prompts, nc_tokenized_prompts):
    grid = (B // BPS,)

    def gspec(m):
        return pl.BlockSpec(
            (1, HALF, D),
            lambda s, idx, m=m: (idx[(2 * BPS * s + m) % B], 0, 0))

    in_specs = (
        [gspec(m) for m in range(2 * BPS)] +
        [gspec(m) for m in range(2 * BPS)] +
        [
            pl.BlockSpec((CLS, 1, D), lambda s, idx: (0, 0, 0)),
            pl.BlockSpec((CLS, SUF, D), lambda s, idx: (0, 0, 0)),
            pl.BlockSpec((1, 1, D), lambda s, idx: (0, 0, 0)),
            pl.BlockSpec((1, SUF, D), lambda s, idx: (0, 0, 0)),
            pl.BlockSpec((ROWS, HALF, D), lambda s, idx: (s, 0, 0)),
            pl.BlockSpec((ROWS, HALF, D), lambda s, idx: (s, 0, 0)),
            pl.BlockSpec((CLS, SEQ), lambda s, idx: (0, 0)),
            pl.BlockSpec((1, SEQ), lambda s, idx: (0, 0)),
        ])
    out_specs = [
        pl.BlockSpec((ROWS, SEQ, D), lambda s, idx: (s, 0, 0)),
        pl.BlockSpec((ROWS, SEQ), lambda s, idx: (s, 0)),
        pl.BlockSpec((ROWS, SEQ, D), lambda s, idx: (s, 0, 0)),
        pl.BlockSpec((ROWS, SEQ), lambda s, idx: (s, 0)),
    ]
    out_shape = [
        jax.ShapeDtypeStruct((B * CLS, SEQ, D), jnp.float32),
        jax.ShapeDtypeStruct((B * CLS, SEQ), jnp.int32),
        jax.ShapeDtypeStruct((POOL, SEQ, D), jnp.float32),
        jax.ShapeDtypeStruct((POOL, SEQ), jnp.int32),
    ]

    grid_spec = pltpu.PrefetchScalarGridSpec(
        num_scalar_prefetch=1,
        grid=grid,
        in_specs=in_specs,
        out_specs=out_specs,
    )
    prompts, tok, nc_prompts, nc_tok = pl.pallas_call(
        _body,
        grid_spec=grid_spec,
        out_shape=out_shape,
    )(indices_g,
      *([global_prompt] * (2 * BPS)), *([attribute_prompt] * (2 * BPS)),
      token_prefix, token_suffix, nc_token_prefix, nc_token_suffix,
      global_prompt, attribute_prompt, tokenized_prompts,
      nc_tokenized_prompts)

    return (prompts, tok, nc_prompts, nc_tok)
